# Initial kernel scaffold; baseline (speedup 1.0000x reference)
import jax
import jax.numpy as jnp
from jax.experimental import pallas as pl


def kernel(x, edge_index, batch, W_in, b_in, Wg0, bg0, Wg1, bg1, Wg2, bg2, gamma0, beta0, gamma1, beta1, gamma2, beta2, Wa1, ba1, Wa2, ba2, Wo1, bo1, Wo2, bo2):
    raise NotImplementedError("write your pallas kernel here")


# speedup vs baseline: 7.7948x; 7.7948x over previous
import functools
import math

import jax
import jax.numpy as jnp
from jax import lax
from jax.experimental import pallas as pl
from jax.experimental.pallas import tpu as pltpu
from jax.experimental.pallas import tpu_sc as plsc

N = 10000
NP = 10240
E = 320000
IN_DIM = 128
HID = 256
HALF = 128
G = 16
NS = 16
ROWS_PER = NP // NS
BLK = 640
GRID = NP // BLK
K = 80
E_PER_SUB = E // NS
E_PER_SUB_DEG = E // (2 * NS)
BNC = 1.0 / math.sqrt(1.0 + 1e-5)

_f32 = jnp.float32


def _deg_counts(dst, ones_c, zeros_c):
    mesh = plsc.VectorSubcoreMesh(core_axis_name="c", subcore_axis_name="s")

    @functools.partial(
        pl.kernel,
        out_type=jax.ShapeDtypeStruct((2, NP, 16), _f32),
        mesh=mesh,
        scratch_types=[
            pltpu.VMEM((K,), jnp.int32),
            pltpu.VMEM((K, 16), _f32),
            pltpu.VMEM_SHARED((NP, 16), _f32),
        ],
    )
    def deg_kernel(dst_hbm, ones_hbm, zeros_hbm, out_hbm, dst_v, ones_v, accd):
        c = lax.axis_index("c")
        s = lax.axis_index("s")
        r0 = s * ROWS_PER

        pltpu.sync_copy(ones_hbm, ones_v)
        pltpu.sync_copy(zeros_hbm, accd.at[pl.ds(r0, ROWS_PER)])
        plsc.subcore_barrier()

        base = (c * NS + s) * E_PER_SUB_DEG

        def body(j, carry):
            b = base + j * K
            pltpu.sync_copy(dst_hbm.at[pl.ds(b, K)], dst_v)
            pltpu.sync_copy(ones_v, accd.at[dst_v], add=True)
            return carry

        lax.fori_loop(0, E_PER_SUB_DEG // K, body, 0)
        plsc.subcore_barrier()

        @pl.when(c == 0)
        def _():
            pltpu.sync_copy(accd.at[pl.ds(r0, ROWS_PER)],
                            out_hbm.at[0, pl.ds(r0, ROWS_PER)])

        @pl.when(c == 1)
        def _():
            pltpu.sync_copy(accd.at[pl.ds(r0, ROWS_PER)],
                            out_hbm.at[1, pl.ds(r0, ROWS_PER)])

    return deg_kernel(dst, ones_c, zeros_c)


def _propagate(s_lo, s_hi, src, dst):
    mesh = plsc.VectorSubcoreMesh(core_axis_name="c", subcore_axis_name="s")

    @functools.partial(
        pl.kernel,
        out_type=(jax.ShapeDtypeStruct((NP, HALF), _f32),
                  jax.ShapeDtypeStruct((NP, HALF), _f32)),
        mesh=mesh,
        scratch_types=[
            pltpu.VMEM((K,), jnp.int32),
            pltpu.VMEM((K,), jnp.int32),
            pltpu.VMEM((K, HALF), _f32),
            pltpu.VMEM_SHARED((NP, HALF), _f32),
            pltpu.SemaphoreType.DMA,
        ],
    )
    def prop_kernel(slo_hbm, shi_hbm, src_hbm, dst_hbm, olo_hbm, ohi_hbm,
                    src_v, dst_v, rows_v, acc, sem):
        c = lax.axis_index("c")
        s = lax.axis_index("s")
        r0 = s * ROWS_PER

        @pl.when(c == 0)
        def _():
            pltpu.sync_copy(slo_hbm.at[pl.ds(r0, ROWS_PER)],
                            acc.at[pl.ds(r0, ROWS_PER)])

        @pl.when(c == 1)
        def _():
            pltpu.sync_copy(shi_hbm.at[pl.ds(r0, ROWS_PER)],
                            acc.at[pl.ds(r0, ROWS_PER)])

        plsc.subcore_barrier()

        base = s * E_PER_SUB

        def body(j, carry):
            b = base + j * K
            pltpu.sync_copy(src_hbm.at[pl.ds(b, K)], src_v)
            pltpu.sync_copy(dst_hbm.at[pl.ds(b, K)], dst_v)

            @pl.when(c == 0)
            def _():
                pltpu.async_copy(slo_hbm.at[src_v], rows_v, sem).wait()

            @pl.when(c == 1)
            def _():
                pltpu.async_copy(shi_hbm.at[src_v], rows_v, sem).wait()

            pltpu.sync_copy(rows_v, acc.at[dst_v], add=True)
            return carry

        lax.fori_loop(0, E_PER_SUB // K, body, 0)
        plsc.subcore_barrier()

        @pl.when(c == 0)
        def _():
            pltpu.sync_copy(acc.at[pl.ds(r0, ROWS_PER)],
                            olo_hbm.at[pl.ds(r0, ROWS_PER)])

        @pl.when(c == 1)
        def _():
            pltpu.sync_copy(acc.at[pl.ds(r0, ROWS_PER)],
                            ohi_hbm.at[pl.ds(r0, ROWS_PER)])

    return prop_kernel(s_lo, s_hi, src, dst)


def _lin_in(x_p, c0, c1, W_in, b_in, Wg0):
    def body(x_ref, c0_ref, c1_ref, win_ref, bin_ref, wg0_ref,
             slo_ref, shi_ref, dinv_ref):
        t = jnp.dot(x_ref[...], win_ref[...], preferred_element_type=_f32)
        t = jnp.maximum(t + bin_ref[...], 0.0)
        deg = c0_ref[...][:, :1] + c1_ref[...][:, :1] + 1.0
        dinv = lax.rsqrt(deg)
        sarr = jnp.dot(t, wg0_ref[...], preferred_element_type=_f32) * dinv
        slo_ref[...] = sarr[:, :HALF]
        shi_ref[...] = sarr[:, HALF:]
        dinv_ref[...] = jnp.broadcast_to(dinv, (BLK, 16))

    return pl.pallas_call(
        body,
        grid=(GRID,),
        in_specs=[
            pl.BlockSpec((BLK, IN_DIM), lambda j: (j, 0)),
            pl.BlockSpec((BLK, 16), lambda j: (j, 0)),
            pl.BlockSpec((BLK, 16), lambda j: (j, 0)),
            pl.BlockSpec((IN_DIM, HID), lambda j: (0, 0)),
            pl.BlockSpec((1, HID), lambda j: (0, 0)),
            pl.BlockSpec((HID, HID), lambda j: (0, 0)),
        ],
        out_specs=[
            pl.BlockSpec((BLK, HALF), lambda j: (j, 0)),
            pl.BlockSpec((BLK, HALF), lambda j: (j, 0)),
            pl.BlockSpec((BLK, 16), lambda j: (j, 0)),
        ],
        out_shape=[
            jax.ShapeDtypeStruct((NP, HALF), _f32),
            jax.ShapeDtypeStruct((NP, HALF), _f32),
            jax.ShapeDtypeStruct((NP, 16), _f32),
        ],
    )(x_p, c0, c1, W_in, b_in, Wg0)


def _post_layer(a_lo, a_hi, dinv16, h_prev, b_l, gamma, beta, W_next):
    res = h_prev is not None

    def body(*refs):
        if res:
            (alo_ref, ahi_ref, dv_ref, hp_ref, b_ref, g_ref, bt_ref, wn_ref,
             h_ref, slo_ref, shi_ref) = refs
        else:
            (alo_ref, ahi_ref, dv_ref, b_ref, g_ref, bt_ref, wn_ref,
             h_ref, slo_ref, shi_ref) = refs
        dinv = dv_ref[...][:, :1]
        a = jnp.concatenate([alo_ref[...], ahi_ref[...]], axis=1)
        h = (a * dinv + b_ref[...]) * (g_ref[...] * BNC) + bt_ref[...]
        h = jnp.maximum(h, 0.0)
        if res:
            h = h + hp_ref[...]
        h_ref[...] = h
        sarr = jnp.dot(h, wn_ref[...], preferred_element_type=_f32) * dinv
        slo_ref[...] = sarr[:, :HALF]
        shi_ref[...] = sarr[:, HALF:]

    in_specs = [
        pl.BlockSpec((BLK, HALF), lambda j: (j, 0)),
        pl.BlockSpec((BLK, HALF), lambda j: (j, 0)),
        pl.BlockSpec((BLK, 16), lambda j: (j, 0)),
    ]
    args = [a_lo, a_hi, dinv16]
    if res:
        in_specs.append(pl.BlockSpec((BLK, HID), lambda j: (j, 0)))
        args.append(h_prev)
    in_specs += [
        pl.BlockSpec((1, HID), lambda j: (0, 0)),
        pl.BlockSpec((1, HID), lambda j: (0, 0)),
        pl.BlockSpec((1, HID), lambda j: (0, 0)),
        pl.BlockSpec((HID, HID), lambda j: (0, 0)),
    ]
    args += [b_l, gamma, beta, W_next]

    return pl.pallas_call(
        body,
        grid=(GRID,),
        in_specs=in_specs,
        out_specs=[
            pl.BlockSpec((BLK, HID), lambda j: (j, 0)),
            pl.BlockSpec((BLK, HALF), lambda j: (j, 0)),
            pl.BlockSpec((BLK, HALF), lambda j: (j, 0)),
        ],
        out_shape=[
            jax.ShapeDtypeStruct((NP, HID), _f32),
            jax.ShapeDtypeStruct((NP, HALF), _f32),
            jax.ShapeDtypeStruct((NP, HALF), _f32),
        ],
    )(*args)


def _pool_head(a_lo, a_hi, dinv16, h1, batch16, b_l, gamma, beta,
               Wa1, ba1, Wa2, ba2, Wo1, bo1, Wo2, bo2):
    def body(alo_ref, ahi_ref, dv_ref, h1_ref, bt_ref,
             b_ref, g_ref, btt_ref, wa1_ref, ba1_ref, wa2_ref, ba2_ref,
             wo1_ref, bo1_ref, wo2_ref, bo2_ref, out_ref,
             sum_acc, mid_acc, attn_acc, max_acc, cnt_acc, z_acc):
        j = pl.program_id(0)

        @pl.when(j == 0)
        def _():
            sum_acc[...] = jnp.zeros((G, HID), _f32)
            mid_acc[...] = jnp.zeros((G, HID), _f32)
            attn_acc[...] = jnp.zeros((G, HID), _f32)
            cnt_acc[...] = jnp.zeros((G, HID), _f32)
            max_acc[...] = jnp.full((G, HID), -jnp.inf, _f32)
            z_acc[0] = 0.0

        dinv = dv_ref[...][:, :1]
        a = jnp.concatenate([alo_ref[...], ahi_ref[...]], axis=1)
        h2 = (a * dinv + b_ref[...]) * (g_ref[...] * BNC) + btt_ref[...]
        h2 = jnp.maximum(h2, 0.0) + h1_ref[...]
        h1b = h1_ref[...]

        bt = bt_ref[...][:, :1]
        validf = (bt < G).astype(_f32)
        iota_g = lax.broadcasted_iota(jnp.int32, (BLK, G), 1)
        Pt = (iota_g == bt).astype(_f32)

        dn = (((0,), (0,)), ((), ()))
        sum_acc[...] += lax.dot_general(Pt, h2, dn,
                                        preferred_element_type=_f32)
        mid_acc[...] += lax.dot_general(Pt, h1b, dn,
                                        preferred_element_type=_f32)

        th = jnp.tanh(jnp.dot(h2, wa1_ref[...],
                              preferred_element_type=_f32) + ba1_ref[...])
        aw = jnp.dot(th, wa2_ref[...], preferred_element_type=_f32) + ba2_ref[...]
        e = jnp.exp(aw) * validf
        attn_acc[...] += lax.dot_general(Pt, h2 * e, dn,
                                         preferred_element_type=_f32)
        z_acc[0] += jnp.sum(e)

        cnt_col = lax.dot_general(Pt, validf, dn,
                                  preferred_element_type=_f32)
        cnt_acc[...] += jnp.broadcast_to(cnt_col, (G, HID))

        rows = []
        for g_idx in range(G):
            mg = jnp.max(jnp.where(bt == g_idx, h2, -jnp.inf),
                         axis=0, keepdims=True)
            rows.append(mg)
        max_acc[...] = jnp.maximum(max_acc[...], jnp.concatenate(rows, axis=0))

        @pl.when(j == GRID - 1)
        def _():
            cnt = jnp.maximum(cnt_acc[...], 1.0)
            mean_p = sum_acc[...] / cnt
            mid_p = mid_acc[...] / cnt
            attn_p = attn_acc[...] / z_acc[0]
            combined = jnp.concatenate(
                [mean_p, max_acc[...], attn_p, mid_p], axis=1)
            r = jnp.maximum(
                jnp.dot(combined, wo1_ref[...], preferred_element_type=_f32)
                + bo1_ref[...], 0.0)
            out_ref[...] = (jnp.dot(r, wo2_ref[...],
                                    preferred_element_type=_f32)
                            + bo2_ref[...])

    return pl.pallas_call(
        body,
        grid=(GRID,),
        in_specs=[
            pl.BlockSpec((BLK, HALF), lambda j: (j, 0)),
            pl.BlockSpec((BLK, HALF), lambda j: (j, 0)),
            pl.BlockSpec((BLK, 16), lambda j: (j, 0)),
            pl.BlockSpec((BLK, HID), lambda j: (j, 0)),
            pl.BlockSpec((BLK, 16), lambda j: (j, 0)),
            pl.BlockSpec((1, HID), lambda j: (0, 0)),
            pl.BlockSpec((1, HID), lambda j: (0, 0)),
            pl.BlockSpec((1, HID), lambda j: (0, 0)),
            pl.BlockSpec((HID, HID), lambda j: (0, 0)),
            pl.BlockSpec((1, HID), lambda j: (0, 0)),
            pl.BlockSpec((HID, 1), lambda j: (0, 0)),
            pl.BlockSpec((1, 1), lambda j: (0, 0)),
            pl.BlockSpec((4 * HID, HID), lambda j: (0, 0)),
            pl.BlockSpec((1, HID), lambda j: (0, 0)),
            pl.BlockSpec((HID, HID), lambda j: (0, 0)),
            pl.BlockSpec((1, HID), lambda j: (0, 0)),
        ],
        out_specs=pl.BlockSpec((G, HID), lambda j: (0, 0)),
        out_shape=jax.ShapeDtypeStruct((G, HID), _f32),
        scratch_shapes=[
            pltpu.VMEM((G, HID), _f32),
            pltpu.VMEM((G, HID), _f32),
            pltpu.VMEM((G, HID), _f32),
            pltpu.VMEM((G, HID), _f32),
            pltpu.VMEM((G, HID), _f32),
            pltpu.SMEM((1,), _f32),
        ],
    )(a_lo, a_hi, dinv16, h1, batch16, b_l, gamma, beta,
      Wa1, ba1, Wa2, ba2, Wo1, bo1, Wo2, bo2)


def kernel(x, edge_index, batch, W_in, b_in, Wg0, bg0, Wg1, bg1, Wg2, bg2,
           gamma0, beta0, gamma1, beta1, gamma2, beta2,
           Wa1, ba1, Wa2, ba2, Wo1, bo1, Wo2, bo2):
    edge = edge_index.astype(jnp.int32)
    src_e, dst_e = edge[0], edge[1]
    x_p = jnp.pad(x, ((0, NP - N), (0, 0)))
    batch_p = jnp.pad(batch.astype(jnp.int32), (0, NP - N),
                      constant_values=G)
    batch16 = jnp.broadcast_to(batch_p[:, None], (NP, 16))

    ones_c = jnp.ones((K, 16), _f32)
    zeros_c = jnp.zeros((ROWS_PER, 16), _f32)

    cnts = _deg_counts(dst_e, ones_c, zeros_c)
    c0, c1 = cnts[0], cnts[1]

    r2 = lambda v: v.reshape(1, -1)
    s0_lo, s0_hi, dinv16 = _lin_in(x_p, c0, c1, W_in, r2(b_in), Wg0)

    a_lo, a_hi = _propagate(s0_lo, s0_hi, src_e, dst_e)
    h0, s1_lo, s1_hi = _post_layer(a_lo, a_hi, dinv16, None,
                                   r2(bg0), r2(gamma0), r2(beta0), Wg1)

    a_lo, a_hi = _propagate(s1_lo, s1_hi, src_e, dst_e)
    h1, s2_lo, s2_hi = _post_layer(a_lo, a_hi, dinv16, h0,
                                   r2(bg1), r2(gamma1), r2(beta1), Wg2)

    a_lo, a_hi = _propagate(s2_lo, s2_hi, src_e, dst_e)
    out = _pool_head(a_lo, a_hi, dinv16, h1, batch16,
                     r2(bg2), r2(gamma2), r2(beta2),
                     Wa1, r2(ba1), Wa2.reshape(HID, 1), ba2.reshape(1, 1),
                     Wo1, r2(bo1), Wo2, r2(bo2))
    return out
